# baseline (speedup 1.0000x reference)
import jax
import jax.numpy as jnp
from jax.experimental import pallas as pl
from jax.experimental.pallas import tpu as pltpu

T, B = 32, 64
H, W = 16, 16
FEAT = 32
CNN_DIM = 512
N_ACT = 5
P = H * W
HC = 128


def _conv_relu_nhwc(x, w, b, stride):
    y = jax.lax.conv_general_dilated(
        x.astype(jnp.bfloat16), w.astype(jnp.bfloat16), (stride, stride),
        'VALID', dimension_numbers=('NHWC', 'HWIO', 'NHWC'),
        preferred_element_type=jnp.float32)
    return jax.nn.relu(y + b[None, None, None, :])


G = 8


def _scan_heads_kernel(p_ref,
                       hb_ref, m_ref, s_ref, pf_ref, st0_ref, h10_ref, oh_ref,
                       wc_ref, ww_ref, bw_ref, wp1_ref, bp1_ref, wp2_ref,
                       bp2_ref, tail_ref, bc_ref, wh_ref, bh_ref,
                       out_ref, state_ref, h1s_ref):
    bb = pl.program_id(0)
    wfeat = jnp.dot(hb_ref[:].reshape(G * T, CNN_DIM), ww_ref[:],
                    preferred_element_type=jnp.float32,
                    precision=jax.lax.Precision.HIGHEST) + bw_ref[:]
    h1s = [h10_ref[g] for g in range(G)]
    for t in range(T):
        for g in range(G):
            p = p_ref[bb * G + g, t]
            wsl = wc_ref[p]
            wf_t = wfeat[g * T + t:g * T + t + 1, :]
            h1s[g] = h1s[g] * m_ref[g, :, t:t + 1] + jnp.dot(
                wf_t, wsl, preferred_element_type=jnp.float32,
                precision=jax.lax.Precision.HIGHEST)
            h1s_ref[g * T + t:g * T + t + 1, :] = h1s[g]
    iq = jax.lax.broadcasted_iota(jnp.int32, (P, T), 0)
    for g in range(G):
        ohs = (iq == pf_ref[g]).astype(jnp.float32) * s_ref[g]
        state_ref[g] = st0_ref[g] + jnp.dot(
            ohs, wfeat[g * T:(g + 1) * T, :],
            preferred_element_type=jnp.float32,
            precision=jax.lax.Precision.HIGHEST)
    posh = jnp.maximum(jnp.dot(oh_ref[:].reshape(G * T, 2 * H), wp1_ref[:],
                               preferred_element_type=jnp.float32,
                               precision=jax.lax.Precision.HIGHEST)
                       + bp1_ref[:], 0.0)
    posf = jnp.dot(posh, wp2_ref[:],
                   preferred_element_type=jnp.float32,
                   precision=jax.lax.Precision.HIGHEST) + bp2_ref[:]
    z = h1s_ref[:] + jnp.dot(posf, tail_ref[:],
                             preferred_element_type=jnp.float32,
                             precision=jax.lax.Precision.HIGHEST) + bc_ref[:]
    a = jnp.maximum(z, 0.0)
    out = jnp.dot(a, wh_ref[:], preferred_element_type=jnp.float32,
                  precision=jax.lax.Precision.HIGHEST) + bh_ref[:]
    out_ref[:] = out.reshape(G, T, 8)


def kernel(image, done, state0, w1, b1, w2, b2, w3, b3, wfc, bfc, Ww, bw,
           wp1, bp1, wp2, bp2, wpo1, bpo1, wpo2, bpo2, wv1, bv1, wv2, bv2,
           position):
    f32 = jnp.float32
    w1t = jnp.transpose(w1, (2, 3, 1, 0))
    w2t = jnp.transpose(w2, (2, 3, 1, 0))
    w3t = jnp.transpose(w3, (2, 3, 1, 0))
    x = _conv_relu_nhwc(image / 255.0, w1t, b1, 4)
    x = _conv_relu_nhwc(x, w2t, b2, 2)
    x = _conv_relu_nhwc(x, w3t, b3, 1)
    wfc_r = wfc.reshape(64, 7, 7, CNN_DIM).transpose(1, 2, 0, 3)
    wfc_r = wfc_r.reshape(7 * 7 * 64, CNN_DIM)
    h = jax.nn.relu(x.reshape(T * B, 7 * 7 * 64) @ wfc_r + bfc)
    hb = h.reshape(T, B, CNN_DIM).transpose(1, 0, 2)

    pos_y = position[:, 0]
    pos_x = position[:, 1]
    p_bt = (pos_y * W + pos_x).reshape(T, B).transpose()
    m = 1.0 - done
    cp_rev = jnp.cumprod(m[::-1], axis=0)
    s_suf = jnp.concatenate([cp_rev[::-1][1:], jnp.ones((1, B), f32)], axis=0)
    prod_all = cp_rev[-1]
    m_bt = m.transpose().reshape(B, 1, T)
    s_bt = s_suf.transpose().reshape(B, 1, T)
    pf_b = p_bt.reshape(B, 1, T)
    st0 = state0.transpose(0, 2, 3, 1).reshape(B, P, FEAT)
    st0 = st0 * prod_all[:, None, None]
    wc_flat = jnp.concatenate([wpo1[:P * FEAT], wv1[:P * FEAT]], axis=1)
    wc = wc_flat.reshape(FEAT, P, HC).transpose(1, 0, 2)
    h10 = (state0.reshape(B, P * FEAT) @ wc_flat).reshape(B, 1, HC)
    tail_c = jnp.concatenate([wpo1[P * FEAT:], wv1[P * FEAT:]], axis=1)
    bc = jnp.concatenate([bpo1, bv1]).reshape(1, HC)
    wh = jnp.zeros((HC, 8), f32)
    wh = wh.at[:64, :N_ACT].set(wpo2).at[64:, N_ACT:N_ACT + 1].set(wv2)
    bh = jnp.concatenate([bpo2, bv2, jnp.zeros((2,), f32)]).reshape(1, 8)
    oh = jnp.concatenate([jax.nn.one_hot(pos_y, H, dtype=f32),
                          jax.nn.one_hot(pos_x, W, dtype=f32)], axis=1)
    oh_b = oh.reshape(T, B, 2 * H).transpose(1, 0, 2)

    grid_spec = pltpu.PrefetchScalarGridSpec(
        num_scalar_prefetch=1,
        grid=(B // G,),
        in_specs=[
            pl.BlockSpec((G, T, CNN_DIM), lambda b, *_: (b, 0, 0)),
            pl.BlockSpec((G, 1, T), lambda b, *_: (b, 0, 0)),
            pl.BlockSpec((G, 1, T), lambda b, *_: (b, 0, 0)),
            pl.BlockSpec((G, 1, T), lambda b, *_: (b, 0, 0)),
            pl.BlockSpec((G, P, FEAT), lambda b, *_: (b, 0, 0)),
            pl.BlockSpec((G, 1, HC), lambda b, *_: (b, 0, 0)),
            pl.BlockSpec((G, T, 2 * H), lambda b, *_: (b, 0, 0)),
            pl.BlockSpec((P, FEAT, HC), lambda b, *_: (0, 0, 0)),
            pl.BlockSpec((CNN_DIM, FEAT), lambda b, *_: (0, 0)),
            pl.BlockSpec((1, FEAT), lambda b, *_: (0, 0)),
            pl.BlockSpec((2 * H, 64), lambda b, *_: (0, 0)),
            pl.BlockSpec((1, 64), lambda b, *_: (0, 0)),
            pl.BlockSpec((64, 64), lambda b, *_: (0, 0)),
            pl.BlockSpec((1, 64), lambda b, *_: (0, 0)),
            pl.BlockSpec((64, HC), lambda b, *_: (0, 0)),
            pl.BlockSpec((1, HC), lambda b, *_: (0, 0)),
            pl.BlockSpec((HC, 8), lambda b, *_: (0, 0)),
            pl.BlockSpec((1, 8), lambda b, *_: (0, 0)),
        ],
        out_specs=[
            pl.BlockSpec((G, T, 8), lambda b, *_: (b, 0, 0)),
            pl.BlockSpec((G, P, FEAT), lambda b, *_: (b, 0, 0)),
        ],
        scratch_shapes=[pltpu.VMEM((G * T, HC), f32)],
    )
    out6, state_acc = pl.pallas_call(
        _scan_heads_kernel,
        grid_spec=grid_spec,
        out_shape=[jax.ShapeDtypeStruct((B, T, 8), f32),
                   jax.ShapeDtypeStruct((B, P, FEAT), f32)],
        compiler_params=pltpu.CompilerParams(
            dimension_semantics=('parallel',)),
    )(p_bt, hb, m_bt, s_bt, pf_b, st0, h10, oh_b, wc, Ww, bw.reshape(1, FEAT),
      wp1, bp1.reshape(1, 64), wp2, bp2.reshape(1, 64), tail_c, bc, wh, bh)

    out = out6.transpose(1, 0, 2).reshape(T * B, 8)
    logits = out[:, :N_ACT]
    v = out[:, N_ACT:N_ACT + 1]
    state = state_acc.reshape(B, H, W, FEAT).transpose(0, 3, 1, 2)
    return logits, v, state


# speedup vs baseline: 1.0553x; 1.0553x over previous
import jax
import jax.numpy as jnp
from jax.experimental import pallas as pl
from jax.experimental.pallas import tpu as pltpu

T, B = 32, 64
H, W = 16, 16
FEAT = 32
CNN_DIM = 512
N_ACT = 5
P = H * W
HC = 128


def _conv_relu_nhwc(x, w, b, stride):
    y = jax.lax.conv_general_dilated(
        x.astype(jnp.bfloat16), w.astype(jnp.bfloat16), (stride, stride),
        'VALID', dimension_numbers=('NHWC', 'HWIO', 'NHWC'),
        preferred_element_type=jnp.float32)
    return jax.nn.relu(y + b[None, None, None, :])


G = 8


def _scan_heads_kernel(p_ref,
                       hb_ref, m_ref, s_ref, st0_ref, h10_ref, oh_ref,
                       wc_ref, ww_ref, bw_ref, wp1_ref, bp1_ref, wp2_ref,
                       bp2_ref, tail_ref, bc_ref, wh_ref, bh_ref,
                       out_ref, state_ref, h1s_ref):
    bb = pl.program_id(0)
    wfeat = jnp.dot(hb_ref[:].reshape(G * T, CNN_DIM), ww_ref[:],
                    preferred_element_type=jnp.float32,
                    precision=jax.lax.Precision.HIGHEST) + bw_ref[:]
    state_ref[:] = st0_ref[:]
    h1s = [h10_ref[g] for g in range(G)]
    for t in range(T):
        for g in range(G):
            p = p_ref[bb * G + g, t]
            wsl = wc_ref[p]
            wf_t = wfeat[g * T + t:g * T + t + 1, :]
            h1s[g] = h1s[g] * m_ref[g, :, t:t + 1] + jnp.dot(
                wf_t, wsl, preferred_element_type=jnp.float32)
            h1s_ref[g * T + t:g * T + t + 1, :] = h1s[g]
            base = pl.multiple_of((p >> 3) << 3, 8)
            rowsel = (jax.lax.broadcasted_iota(jnp.int32, (8, 1), 0)
                      == (p - base))
            add = jnp.where(rowsel, 1.0, 0.0) * (wf_t * s_ref[g, :, t:t + 1])
            state_ref[g, pl.ds(base, 8), :] = (
                state_ref[g, pl.ds(base, 8), :] + add)
    posh = jnp.maximum(jnp.dot(oh_ref[:].reshape(G * T, 2 * H), wp1_ref[:],
                               preferred_element_type=jnp.float32,
                               precision=jax.lax.Precision.HIGHEST)
                       + bp1_ref[:], 0.0)
    posf = jnp.dot(posh, wp2_ref[:],
                   preferred_element_type=jnp.float32,
                   precision=jax.lax.Precision.HIGHEST) + bp2_ref[:]
    z = h1s_ref[:] + jnp.dot(posf, tail_ref[:],
                             preferred_element_type=jnp.float32,
                             precision=jax.lax.Precision.HIGHEST) + bc_ref[:]
    a = jnp.maximum(z, 0.0)
    out = jnp.dot(a, wh_ref[:], preferred_element_type=jnp.float32,
                  precision=jax.lax.Precision.HIGHEST) + bh_ref[:]
    out_ref[:] = out.reshape(G, T, 8)


def kernel(image, done, state0, w1, b1, w2, b2, w3, b3, wfc, bfc, Ww, bw,
           wp1, bp1, wp2, bp2, wpo1, bpo1, wpo2, bpo2, wv1, bv1, wv2, bv2,
           position):
    f32 = jnp.float32
    w1t = jnp.transpose(w1, (2, 3, 1, 0))
    w2t = jnp.transpose(w2, (2, 3, 1, 0))
    w3t = jnp.transpose(w3, (2, 3, 1, 0))
    x = _conv_relu_nhwc(image / 255.0, w1t, b1, 4)
    x = _conv_relu_nhwc(x, w2t, b2, 2)
    x = _conv_relu_nhwc(x, w3t, b3, 1)
    wfc_r = wfc.reshape(64, 7, 7, CNN_DIM).transpose(1, 2, 0, 3)
    wfc_r = wfc_r.reshape(7 * 7 * 64, CNN_DIM)
    h = jax.nn.relu(x.reshape(T * B, 7 * 7 * 64) @ wfc_r + bfc)
    hb = h.reshape(T, B, CNN_DIM).transpose(1, 0, 2)

    pos_y = position[:, 0]
    pos_x = position[:, 1]
    p_bt = (pos_y * W + pos_x).reshape(T, B).transpose()
    m = 1.0 - done
    cp_rev = jnp.cumprod(m[::-1], axis=0)
    s_suf = jnp.concatenate([cp_rev[::-1][1:], jnp.ones((1, B), f32)], axis=0)
    prod_all = cp_rev[-1]
    m_bt = m.transpose().reshape(B, 1, T)
    s_bt = s_suf.transpose().reshape(B, 1, T)
    st0 = state0.transpose(0, 2, 3, 1).reshape(B, P, FEAT)
    st0 = st0 * prod_all[:, None, None]
    wc_flat = jnp.concatenate([wpo1[:P * FEAT], wv1[:P * FEAT]], axis=1)
    wc = wc_flat.reshape(FEAT, P, HC).transpose(1, 0, 2)
    h10 = (state0.reshape(B, P * FEAT) @ wc_flat).reshape(B, 1, HC)
    tail_c = jnp.concatenate([wpo1[P * FEAT:], wv1[P * FEAT:]], axis=1)
    bc = jnp.concatenate([bpo1, bv1]).reshape(1, HC)
    wh = jnp.zeros((HC, 8), f32)
    wh = wh.at[:64, :N_ACT].set(wpo2).at[64:, N_ACT:N_ACT + 1].set(wv2)
    bh = jnp.concatenate([bpo2, bv2, jnp.zeros((2,), f32)]).reshape(1, 8)
    oh = jnp.concatenate([jax.nn.one_hot(pos_y, H, dtype=f32),
                          jax.nn.one_hot(pos_x, W, dtype=f32)], axis=1)
    oh_b = oh.reshape(T, B, 2 * H).transpose(1, 0, 2)

    grid_spec = pltpu.PrefetchScalarGridSpec(
        num_scalar_prefetch=1,
        grid=(B // G,),
        in_specs=[
            pl.BlockSpec((G, T, CNN_DIM), lambda b, *_: (b, 0, 0)),
            pl.BlockSpec((G, 1, T), lambda b, *_: (b, 0, 0)),
            pl.BlockSpec((G, 1, T), lambda b, *_: (b, 0, 0)),
            pl.BlockSpec((G, P, FEAT), lambda b, *_: (b, 0, 0)),
            pl.BlockSpec((G, 1, HC), lambda b, *_: (b, 0, 0)),
            pl.BlockSpec((G, T, 2 * H), lambda b, *_: (b, 0, 0)),
            pl.BlockSpec((P, FEAT, HC), lambda b, *_: (0, 0, 0)),
            pl.BlockSpec((CNN_DIM, FEAT), lambda b, *_: (0, 0)),
            pl.BlockSpec((1, FEAT), lambda b, *_: (0, 0)),
            pl.BlockSpec((2 * H, 64), lambda b, *_: (0, 0)),
            pl.BlockSpec((1, 64), lambda b, *_: (0, 0)),
            pl.BlockSpec((64, 64), lambda b, *_: (0, 0)),
            pl.BlockSpec((1, 64), lambda b, *_: (0, 0)),
            pl.BlockSpec((64, HC), lambda b, *_: (0, 0)),
            pl.BlockSpec((1, HC), lambda b, *_: (0, 0)),
            pl.BlockSpec((HC, 8), lambda b, *_: (0, 0)),
            pl.BlockSpec((1, 8), lambda b, *_: (0, 0)),
        ],
        out_specs=[
            pl.BlockSpec((G, T, 8), lambda b, *_: (b, 0, 0)),
            pl.BlockSpec((G, P, FEAT), lambda b, *_: (b, 0, 0)),
        ],
        scratch_shapes=[pltpu.VMEM((G * T, HC), f32)],
    )
    out6, state_acc = pl.pallas_call(
        _scan_heads_kernel,
        grid_spec=grid_spec,
        out_shape=[jax.ShapeDtypeStruct((B, T, 8), f32),
                   jax.ShapeDtypeStruct((B, P, FEAT), f32)],
        compiler_params=pltpu.CompilerParams(
            dimension_semantics=('parallel',)),
    )(p_bt, hb, m_bt, s_bt, st0, h10, oh_b, wc, Ww, bw.reshape(1, FEAT),
      wp1, bp1.reshape(1, 64), wp2, bp2.reshape(1, 64), tail_c, bc, wh, bh)

    out = out6.transpose(1, 0, 2).reshape(T * B, 8)
    logits = out[:, :N_ACT]
    v = out[:, N_ACT:N_ACT + 1]
    state = state_acc.reshape(B, H, W, FEAT).transpose(0, 3, 1, 2)
    return logits, v, state


# speedup vs baseline: 1.0863x; 1.0293x over previous
import jax
import jax.numpy as jnp
from jax.experimental import pallas as pl
from jax.experimental.pallas import tpu as pltpu

T, B = 32, 64
H, W = 16, 16
FEAT = 32
CNN_DIM = 512
N_ACT = 5
P = H * W
HC = 128


def _conv_relu_nhwc(x, w, b, stride):
    y = jax.lax.conv_general_dilated(
        x.astype(jnp.bfloat16), w.astype(jnp.bfloat16), (stride, stride),
        'VALID', dimension_numbers=('NHWC', 'HWIO', 'NHWC'),
        preferred_element_type=jnp.float32)
    return jax.nn.relu(y + b[None, None, None, :])


G = 8


def _scan_heads_kernel(p_ref,
                       hb_ref, m_ref, s_ref, st0_ref, h10_ref, oh_ref,
                       wc_ref, ww_ref, bw_ref, wp1_ref, bp1_ref, wp2_ref,
                       bp2_ref, tail_ref, bc_ref, wh_ref, bh_ref,
                       out_ref, state_ref, h1s_ref):
    bb = pl.program_id(0)
    wfeat = jnp.dot(hb_ref[:].reshape(G * T, CNN_DIM), ww_ref[:],
                    preferred_element_type=jnp.float32) + bw_ref[:]
    state_ref[:] = st0_ref[:]
    h1s = [h10_ref[g] for g in range(G)]
    for t in range(T):
        for g in range(G):
            p = p_ref[bb * G + g, t]
            wsl = wc_ref[p]
            wf_t = wfeat[g * T + t:g * T + t + 1, :]
            h1s[g] = h1s[g] * m_ref[g, :, t:t + 1] + jnp.dot(
                wf_t, wsl, preferred_element_type=jnp.float32)
            h1s_ref[g * T + t:g * T + t + 1, :] = h1s[g]
            base = pl.multiple_of((p >> 3) << 3, 8)
            rowsel = (jax.lax.broadcasted_iota(jnp.int32, (8, 1), 0)
                      == (p - base))
            add = jnp.where(rowsel, 1.0, 0.0) * (wf_t * s_ref[g, :, t:t + 1])
            state_ref[g, pl.ds(base, 8), :] = (
                state_ref[g, pl.ds(base, 8), :] + add)
    posh = jnp.maximum(jnp.dot(oh_ref[:].reshape(G * T, 2 * H), wp1_ref[:],
                               preferred_element_type=jnp.float32)
                       + bp1_ref[:], 0.0)
    posf = jnp.dot(posh, wp2_ref[:],
                   preferred_element_type=jnp.float32) + bp2_ref[:]
    z = h1s_ref[:] + jnp.dot(posf, tail_ref[:],
                             preferred_element_type=jnp.float32) + bc_ref[:]
    a = jnp.maximum(z, 0.0)
    out = jnp.dot(a, wh_ref[:], preferred_element_type=jnp.float32) + bh_ref[:]
    out_ref[:] = out.reshape(G, T, 8)


def kernel(image, done, state0, w1, b1, w2, b2, w3, b3, wfc, bfc, Ww, bw,
           wp1, bp1, wp2, bp2, wpo1, bpo1, wpo2, bpo2, wv1, bv1, wv2, bv2,
           position):
    f32 = jnp.float32
    w1t = jnp.transpose(w1, (2, 3, 1, 0))
    w2t = jnp.transpose(w2, (2, 3, 1, 0))
    w3t = jnp.transpose(w3, (2, 3, 1, 0))
    x = _conv_relu_nhwc(image / 255.0, w1t, b1, 4)
    x = _conv_relu_nhwc(x, w2t, b2, 2)
    x = _conv_relu_nhwc(x, w3t, b3, 1)
    wfc_r = wfc.reshape(64, 7, 7, CNN_DIM).transpose(1, 2, 0, 3)
    wfc_r = wfc_r.reshape(7 * 7 * 64, CNN_DIM)
    h = jax.nn.relu(x.reshape(T * B, 7 * 7 * 64) @ wfc_r + bfc)
    hb = h.reshape(T, B, CNN_DIM).transpose(1, 0, 2)

    pos_y = position[:, 0]
    pos_x = position[:, 1]
    p_bt = (pos_y * W + pos_x).reshape(T, B).transpose()
    m = 1.0 - done
    cp_rev = jnp.cumprod(m[::-1], axis=0)
    s_suf = jnp.concatenate([cp_rev[::-1][1:], jnp.ones((1, B), f32)], axis=0)
    prod_all = cp_rev[-1]
    m_bt = m.transpose().reshape(B, 1, T)
    s_bt = s_suf.transpose().reshape(B, 1, T)
    st0 = state0.transpose(0, 2, 3, 1).reshape(B, P, FEAT)
    st0 = st0 * prod_all[:, None, None]
    wc_flat = jnp.concatenate([wpo1[:P * FEAT], wv1[:P * FEAT]], axis=1)
    wc = wc_flat.reshape(FEAT, P, HC).transpose(1, 0, 2)
    h10 = (state0.reshape(B, P * FEAT) @ wc_flat).reshape(B, 1, HC)
    tail_c = jnp.concatenate([wpo1[P * FEAT:], wv1[P * FEAT:]], axis=1)
    bc = jnp.concatenate([bpo1, bv1]).reshape(1, HC)
    wh = jnp.zeros((HC, 8), f32)
    wh = wh.at[:64, :N_ACT].set(wpo2).at[64:, N_ACT:N_ACT + 1].set(wv2)
    bh = jnp.concatenate([bpo2, bv2, jnp.zeros((2,), f32)]).reshape(1, 8)
    oh = jnp.concatenate([jax.nn.one_hot(pos_y, H, dtype=f32),
                          jax.nn.one_hot(pos_x, W, dtype=f32)], axis=1)
    oh_b = oh.reshape(T, B, 2 * H).transpose(1, 0, 2)

    grid_spec = pltpu.PrefetchScalarGridSpec(
        num_scalar_prefetch=1,
        grid=(B // G,),
        in_specs=[
            pl.BlockSpec((G, T, CNN_DIM), lambda b, *_: (b, 0, 0)),
            pl.BlockSpec((G, 1, T), lambda b, *_: (b, 0, 0)),
            pl.BlockSpec((G, 1, T), lambda b, *_: (b, 0, 0)),
            pl.BlockSpec((G, P, FEAT), lambda b, *_: (b, 0, 0)),
            pl.BlockSpec((G, 1, HC), lambda b, *_: (b, 0, 0)),
            pl.BlockSpec((G, T, 2 * H), lambda b, *_: (b, 0, 0)),
            pl.BlockSpec((P, FEAT, HC), lambda b, *_: (0, 0, 0)),
            pl.BlockSpec((CNN_DIM, FEAT), lambda b, *_: (0, 0)),
            pl.BlockSpec((1, FEAT), lambda b, *_: (0, 0)),
            pl.BlockSpec((2 * H, 64), lambda b, *_: (0, 0)),
            pl.BlockSpec((1, 64), lambda b, *_: (0, 0)),
            pl.BlockSpec((64, 64), lambda b, *_: (0, 0)),
            pl.BlockSpec((1, 64), lambda b, *_: (0, 0)),
            pl.BlockSpec((64, HC), lambda b, *_: (0, 0)),
            pl.BlockSpec((1, HC), lambda b, *_: (0, 0)),
            pl.BlockSpec((HC, 8), lambda b, *_: (0, 0)),
            pl.BlockSpec((1, 8), lambda b, *_: (0, 0)),
        ],
        out_specs=[
            pl.BlockSpec((G, T, 8), lambda b, *_: (b, 0, 0)),
            pl.BlockSpec((G, P, FEAT), lambda b, *_: (b, 0, 0)),
        ],
        scratch_shapes=[pltpu.VMEM((G * T, HC), f32)],
    )
    out6, state_acc = pl.pallas_call(
        _scan_heads_kernel,
        grid_spec=grid_spec,
        out_shape=[jax.ShapeDtypeStruct((B, T, 8), f32),
                   jax.ShapeDtypeStruct((B, P, FEAT), f32)],
        compiler_params=pltpu.CompilerParams(
            dimension_semantics=('parallel',)),
    )(p_bt, hb, m_bt, s_bt, st0, h10, oh_b, wc, Ww, bw.reshape(1, FEAT),
      wp1, bp1.reshape(1, 64), wp2, bp2.reshape(1, 64), tail_c, bc, wh, bh)

    out = out6.transpose(1, 0, 2).reshape(T * B, 8)
    logits = out[:, :N_ACT]
    v = out[:, N_ACT:N_ACT + 1]
    state = state_acc.reshape(B, H, W, FEAT).transpose(0, 3, 1, 2)
    return logits, v, state
